# baseline (speedup 1.0000x reference)
import functools

import numpy as np
import jax
import jax.numpy as jnp
from jax import lax
from jax.experimental import pallas as pl
from jax.experimental.pallas import tpu as pltpu
from jax.experimental.pallas import tpu_sc as plsc

N_TOTAL = 100000
D_MODEL = 2048
D_EMB = 128
BATCH = 1024
M_NOISE = 4096
GAMMA = 0.5
TAU = 0.07
Z = np.float32(2876934.2 / 1281167 * N_TOTAL)
C0E = np.float32(M_NOISE * (1.0 / N_TOTAL) + 1e-07)
LOG_C0 = np.float32(np.log(M_NOISE * (1.0 / N_TOTAL)))

NC, NS = 2, 16
NW = NC * NS
NPAIR = BATCH * M_NOISE
NP_W = NPAIR // NW
CS = 8192
NCH = NP_W // CS
ROWS_W = BATCH // NW


def _tf2x32(k1, k2, x0, x1):
    k1 = np.uint32(k1)
    k2 = np.uint32(k2)
    ks = (k1, k2, np.uint32(k1 ^ k2 ^ np.uint32(0x1BD11BDA)))
    rot = (np.array([13, 15, 26, 6]), np.array([17, 29, 16, 24]))
    x0 = x0.astype(np.uint32) + ks[0]
    x1 = x1.astype(np.uint32) + ks[1]

    def rl(x, d):
        return (x << np.uint32(d)) | (x >> np.uint32(32 - d))

    for i in range(5):
        for r in rot[i % 2]:
            x0 = x0 + x1
            x1 = rl(x1, r)
            x1 = x0 ^ x1
        x0 = x0 + ks[(i + 1) % 3]
        x1 = x1 + ks[(i + 2) % 3] + np.uint32(i + 1)
    return x0, x1


def _np_randint_fixed_key(shape, n_total, seed=12345):
    size = int(np.prod(shape))
    b1, b2 = _tf2x32(np.uint32(seed >> 32), np.uint32(seed & 0xFFFFFFFF),
                     np.zeros(2, np.uint32), np.arange(2, dtype=np.uint32))
    subkeys = [(b1[i], b2[i]) for i in range(2)]

    def bits(key):
        a, b = _tf2x32(key[0], key[1], np.zeros(size, np.uint32),
                       np.arange(size, dtype=np.uint32))
        return a ^ b

    higher, lower = bits(subkeys[0]), bits(subkeys[1])
    span = np.uint32(n_total)
    with np.errstate(over="ignore"):
        m0 = np.uint32(2 ** 16) % span
        mult = (m0 * m0) % span
        off = ((higher % span) * mult + (lower % span)) % span
    return off.astype(np.int32).reshape(shape)


_RIDX = _np_randint_fixed_key((BATCH, M_NOISE), N_TOTAL)

_G = 2
_KR = N_TOTAL // _G

_KB = 2000
_HB = _KB // 2
_k_flat = _RIDX.reshape(-1).astype(np.int64)
_i_flat = np.repeat(np.arange(BATCH, dtype=np.int64), M_NOISE)
_GROUPS = []
for _g in range(_G):
    _kk = _k_flat - _g * _KR
    _r = _kk % _KB
    _gsel = (_kk >= 0) & (_kk < _KR)
    _parts = []
    _bufs = []
    for _half in (0, 1):
        _sel = _gsel & ((_r >= _HB) == bool(_half))
        _widx = ((_kk[_sel] // _KB) * (_HB * BATCH)
                 + (_r[_sel] - _HB * _half) * BATCH
                 + _i_flat[_sel]).astype(np.int32)
        _size = int(_widx.size)
        _padded = -(-_size // (NW * 32)) * (NW * 32)
        _widx = np.concatenate(
            [_widx, np.full(_padded - _size, _widx[0], np.int32)])
        _parts.append((_size, _padded))
        _bufs.append(_widx)
    _GROUPS.append((np.concatenate(_bufs), _parts))


def _emb_body(o_ref, w_ref, b_ref, emb_ref, embh_ref):
    x = lax.dot_general(o_ref[...], w_ref[...], (((1,), (1,)), ((), ())),
                        preferred_element_type=jnp.float32,
                        precision=lax.Precision.HIGHEST)
    x = x + b_ref[...]
    e = x / jnp.sqrt(jnp.sum(x * x, axis=1, keepdims=True))
    emb_ref[...] = e
    embh_ref[...] = e.astype(jnp.bfloat16)


def _emb_kernel(outputs, W, b2):
    return pl.pallas_call(
        _emb_body,
        out_shape=(jax.ShapeDtypeStruct((BATCH, D_EMB), jnp.float32),
                   jax.ShapeDtypeStruct((BATCH, D_EMB), jnp.bfloat16)),
    )(outputs, W, b2)


def _score_body(mb_ref, embh_ref, s_ref):
    s = lax.dot_general(
        mb_ref[...].astype(jnp.bfloat16), embh_ref[...],
        (((1,), (1,)), ((), ())), preferred_element_type=jnp.float32)
    b32 = lax.bitcast_convert_type(s, jnp.uint32)
    rb = (b32 + jnp.uint32(0x7FFF) + ((b32 >> jnp.uint32(16))
                                      & jnp.uint32(1))) >> jnp.uint32(16)
    lo = lax.slice(rb, (0, 0), (_HB, BATCH))
    hi = lax.slice(rb, (_HB, 0), (_KB, BATCH))
    w = lo | (hi << jnp.uint32(16))
    s_ref[...] = w.reshape(_HB * BATCH)


def _score_kernel(memory_bank, embh, g):
    off = g * (_KR // _KB)
    return pl.pallas_call(
        _score_body,
        grid=(_KR // _KB,),
        in_specs=[pl.BlockSpec((_KB, D_EMB), lambda i: (off + i, 0)),
                  pl.BlockSpec((BATCH, D_EMB), lambda i: (0, 0))],
        out_specs=pl.BlockSpec((_HB * BATCH,), lambda i: (i,)),
        out_shape=jax.ShapeDtypeStruct((_KR // _KB * _HB * BATCH,),
                                       jnp.uint32),
    )(memory_bank, embh)


@functools.lru_cache(maxsize=None)
def _noise_gather_kernel(npw):
    fc, tail = divmod(npw, CS)
    mesh = plsc.VectorSubcoreMesh(core_axis_name="c", subcore_axis_name="s")
    scratch = [
        pltpu.VMEM((CS,), jnp.int32),
        pltpu.VMEM((CS,), jnp.uint32),
        pltpu.SemaphoreType.DMA,
    ]
    if tail:
        scratch += [pltpu.VMEM((tail,), jnp.int32),
                    pltpu.VMEM((tail,), jnp.uint32)]

    @functools.partial(
        pl.kernel,
        mesh=mesh,
        out_type=jax.ShapeDtypeStruct((npw * NW,), jnp.uint32),
        scratch_types=scratch,
    )
    def _noise_gather(s_hbm, gidx_hbm, out_hbm, idx_v, val_v, sem,
                      *tail_bufs):
        wid = lax.axis_index("s") * NC + lax.axis_index("c")
        base0 = wid * npw

        def chunk(ci, carry):
            base = base0 + ci * CS
            pltpu.sync_copy(gidx_hbm.at[pl.ds(base, CS)], idx_v)
            pltpu.async_copy(s_hbm.at[idx_v], val_v, sem).wait()
            pltpu.sync_copy(val_v, out_hbm.at[pl.ds(base, CS)])
            return carry

        if fc:
            lax.fori_loop(0, fc, chunk, 0)
        if tail:
            ti, tv = tail_bufs
            base = base0 + fc * CS
            pltpu.sync_copy(gidx_hbm.at[pl.ds(base, tail)], ti)
            pltpu.async_copy(s_hbm.at[ti], tv, sem).wait()
            pltpu.sync_copy(tv, out_hbm.at[pl.ds(base, tail)])

    return _noise_gather


@functools.lru_cache(maxsize=None)
def _row_gather_kernel():
    mesh = plsc.VectorSubcoreMesh(core_axis_name="c", subcore_axis_name="s")

    @functools.partial(
        pl.kernel,
        mesh=mesh,
        out_type=jax.ShapeDtypeStruct((BATCH, D_EMB), jnp.float32),
        scratch_types=[
            pltpu.VMEM((ROWS_W,), jnp.int32),
            pltpu.VMEM((ROWS_W, D_EMB), jnp.float32),
            pltpu.SemaphoreType.DMA,
        ],
    )
    def _row_gather(mb_hbm, idx_hbm, out_hbm, idx_v, rows_v, sem):
        wid = lax.axis_index("s") * NC + lax.axis_index("c")
        base = wid * ROWS_W
        pltpu.sync_copy(idx_hbm.at[pl.ds(base, ROWS_W)], idx_v)
        pltpu.async_copy(mb_hbm.at[idx_v], rows_v, sem).wait()
        pltpu.sync_copy(rows_v, out_hbm.at[pl.ds(base, ROWS_W)])

    return _row_gather


def _noise_term(x):
    return LOG_C0 - jnp.log(jnp.exp(x / TAU) / Z + C0E)


def _final_body(*refs):
    emb_ref, md_ref = refs[0], refs[1]
    g_refs = refs[2:2 + _G]
    entries_ref, sums_ref = refs[2 + _G], refs[3 + _G]
    e = emb_ref[...]
    md = md_ref[...]
    data_ip = jnp.sum(e * md, axis=1)
    dp = jnp.exp(data_ip / TAU) / Z
    sum_ld = jnp.sum(jnp.log(dp) - jnp.log(dp + C0E))
    sum_ln = jnp.float32(0.0)
    lane = lax.broadcasted_iota(jnp.int32, (128,), 0)
    for (_, parts), g_ref in zip(_GROUPS, g_refs):
        w = g_ref[...]
        off = 0
        for par, (size, padded) in enumerate(parts):
            wp = lax.slice(w, (off,), (off + padded,))
            if par == 0:
                x = lax.bitcast_convert_type(wp << jnp.uint32(16),
                                             jnp.float32)
            else:
                x = lax.bitcast_convert_type(
                    wp & jnp.uint32(0xFFFF0000), jnp.float32)
            s = jnp.sum(_noise_term(x))
            if padded > size:
                seg = lax.slice(x, (0,), (128,))
                v0 = jnp.sum(jnp.where(lane == 0, seg, 0.0))
                s = s - jnp.float32(padded - size) * _noise_term(v0)
            sum_ln = sum_ln + s
            off += padded
    upd = GAMMA * md + (1.0 - GAMMA) * e
    entries_ref[...] = upd / jnp.sqrt(jnp.sum(upd * upd, axis=1,
                                              keepdims=True))
    row = lax.broadcasted_iota(jnp.int32, (8, 128), 0)
    sums_ref[...] = jnp.where(row == 0, sum_ld, sum_ln)


def _final_kernel(emb, mem_data, noise_groups):
    return pl.pallas_call(
        _final_body,
        out_shape=(jax.ShapeDtypeStruct((BATCH, D_EMB), jnp.float32),
                   jax.ShapeDtypeStruct((8, 128), jnp.float32)),
    )(emb, mem_data, *noise_groups)


def kernel(outputs, indices, memory_bank, W, b):
    emb, embh = _emb_kernel(outputs, W, b.reshape(1, D_EMB))
    mem_data = _row_gather_kernel()(memory_bank, indices)
    noise_groups = []
    for g, (widx, parts) in enumerate(_GROUPS):
        s_g = _score_kernel(memory_bank, embh, g)
        total = sum(pp for _, pp in parts)
        out_g = _noise_gather_kernel(total // NW)(s_g, jnp.asarray(widx))
        noise_groups.append(out_g)
    entries, sums = _final_kernel(emb, mem_data, noise_groups)
    sum_ld = sums[0, 0]
    sum_ln = sums[1, 0]
    loss = jnp.reshape(-(sum_ld + sum_ln) / BATCH, (1,))
    data_loss = jnp.reshape(-sum_ld / BATCH, (1,))
    noise_loss = jnp.reshape(-sum_ln / BATCH, (1,))
    return (loss, entries, data_loss, noise_loss)


# speedup vs baseline: 1.0664x; 1.0664x over previous
import functools

import numpy as np
import jax
import jax.numpy as jnp
from jax import lax
from jax.experimental import pallas as pl
from jax.experimental.pallas import tpu as pltpu
from jax.experimental.pallas import tpu_sc as plsc

N_TOTAL = 100000
D_MODEL = 2048
D_EMB = 128
BATCH = 1024
M_NOISE = 4096
GAMMA = 0.5
TAU = 0.07
Z = np.float32(2876934.2 / 1281167 * N_TOTAL)
C0E = np.float32(M_NOISE * (1.0 / N_TOTAL) + 1e-07)
LOG_C0 = np.float32(np.log(M_NOISE * (1.0 / N_TOTAL)))

NC, NS = 2, 16
NW = NC * NS
CS = 8192
ROWS_W = BATCH // NW


def _tf2x32(k1, k2, x0, x1):
    k1 = np.uint32(k1)
    k2 = np.uint32(k2)
    ks = (k1, k2, np.uint32(k1 ^ k2 ^ np.uint32(0x1BD11BDA)))
    rot = (np.array([13, 15, 26, 6]), np.array([17, 29, 16, 24]))
    x0 = x0.astype(np.uint32) + ks[0]
    x1 = x1.astype(np.uint32) + ks[1]

    def rl(x, d):
        return (x << np.uint32(d)) | (x >> np.uint32(32 - d))

    for i in range(5):
        for r in rot[i % 2]:
            x0 = x0 + x1
            x1 = rl(x1, r)
            x1 = x0 ^ x1
        x0 = x0 + ks[(i + 1) % 3]
        x1 = x1 + ks[(i + 2) % 3] + np.uint32(i + 1)
    return x0, x1


def _np_randint_fixed_key(shape, n_total, seed=12345):
    size = int(np.prod(shape))
    b1, b2 = _tf2x32(np.uint32(seed >> 32), np.uint32(seed & 0xFFFFFFFF),
                     np.zeros(2, np.uint32), np.arange(2, dtype=np.uint32))
    subkeys = [(b1[i], b2[i]) for i in range(2)]

    def bits(key):
        a, b = _tf2x32(key[0], key[1], np.zeros(size, np.uint32),
                       np.arange(size, dtype=np.uint32))
        return a ^ b

    higher, lower = bits(subkeys[0]), bits(subkeys[1])
    span = np.uint32(n_total)
    with np.errstate(over="ignore"):
        m0 = np.uint32(2 ** 16) % span
        mult = (m0 * m0) % span
        off = ((higher % span) * mult + (lower % span)) % span
    return off.astype(np.int32).reshape(shape)


_RIDX = _np_randint_fixed_key((BATCH, M_NOISE), N_TOTAL)

_G = 4
_KR = N_TOTAL // _G

_k_flat = _RIDX.reshape(-1).astype(np.int64)
_i_flat = np.repeat(np.arange(BATCH, dtype=np.int64), M_NOISE)
_GROUPS = []
for _g in range(_G):
    _sel = (_k_flat >= _g * _KR) & (_k_flat < (_g + 1) * _KR)
    _loc = ((_k_flat[_sel] - _g * _KR) * BATCH
            + _i_flat[_sel]).astype(np.int32)
    _size = int(_loc.size)
    _padded = -(-_size // (NW * 32)) * (NW * 32)
    _loc = np.concatenate(
        [_loc, np.full(_padded - _size, _loc[0], np.int32)])
    _GROUPS.append((_loc, _size, _padded))


def _emb_body(o_ref, w_ref, b_ref, emb_ref, embh_ref):
    x = lax.dot_general(o_ref[...], w_ref[...], (((1,), (1,)), ((), ())),
                        preferred_element_type=jnp.float32,
                        precision=lax.Precision.HIGHEST)
    x = x + b_ref[...]
    e = x / jnp.sqrt(jnp.sum(x * x, axis=1, keepdims=True))
    emb_ref[...] = e
    embh_ref[...] = e.astype(jnp.bfloat16)


def _emb_kernel(outputs, W, b2):
    return pl.pallas_call(
        _emb_body,
        out_shape=(jax.ShapeDtypeStruct((BATCH, D_EMB), jnp.float32),
                   jax.ShapeDtypeStruct((BATCH, D_EMB), jnp.bfloat16)),
    )(outputs, W, b2)


_KB = 1000


def _score_body(mb_ref, embh_ref, s_ref):
    s = lax.dot_general(
        mb_ref[...].astype(jnp.bfloat16), embh_ref[...],
        (((1,), (1,)), ((), ())), preferred_element_type=jnp.float32)
    s_ref[...] = s.reshape(_KB * BATCH)


def _score_kernel(memory_bank, embh, g):
    off = g * (_KR // _KB)
    return pl.pallas_call(
        _score_body,
        grid=(_KR // _KB,),
        in_specs=[pl.BlockSpec((_KB, D_EMB), lambda i: (off + i, 0)),
                  pl.BlockSpec((BATCH, D_EMB), lambda i: (0, 0))],
        out_specs=pl.BlockSpec((_KB * BATCH,), lambda i: (i,)),
        out_shape=jax.ShapeDtypeStruct((_KR * BATCH,), jnp.float32),
    )(memory_bank, embh)


@functools.lru_cache(maxsize=None)
def _noise_gather_kernel(npw):
    fc, tail = divmod(npw, CS)
    mesh = plsc.VectorSubcoreMesh(core_axis_name="c", subcore_axis_name="s")
    scratch = [
        pltpu.VMEM((CS,), jnp.int32),
        pltpu.VMEM((CS,), jnp.float32),
        pltpu.SemaphoreType.DMA,
    ]
    if tail:
        scratch += [pltpu.VMEM((tail,), jnp.int32),
                    pltpu.VMEM((tail,), jnp.float32)]

    @functools.partial(
        pl.kernel,
        mesh=mesh,
        out_type=jax.ShapeDtypeStruct((npw * NW,), jnp.float32),
        scratch_types=scratch,
    )
    def _noise_gather(s_hbm, gidx_hbm, out_hbm, idx_v, val_v, sem,
                      *tail_bufs):
        wid = lax.axis_index("s") * NC + lax.axis_index("c")
        base0 = wid * npw

        def chunk(ci, carry):
            base = base0 + ci * CS
            pltpu.sync_copy(gidx_hbm.at[pl.ds(base, CS)], idx_v)
            pltpu.async_copy(s_hbm.at[idx_v], val_v, sem).wait()
            pltpu.sync_copy(val_v, out_hbm.at[pl.ds(base, CS)])
            return carry

        if fc:
            lax.fori_loop(0, fc, chunk, 0)
        if tail:
            ti, tv = tail_bufs
            base = base0 + fc * CS
            pltpu.sync_copy(gidx_hbm.at[pl.ds(base, tail)], ti)
            pltpu.async_copy(s_hbm.at[ti], tv, sem).wait()
            pltpu.sync_copy(tv, out_hbm.at[pl.ds(base, tail)])

    return _noise_gather


@functools.lru_cache(maxsize=None)
def _row_gather_kernel():
    mesh = plsc.VectorSubcoreMesh(core_axis_name="c", subcore_axis_name="s")

    @functools.partial(
        pl.kernel,
        mesh=mesh,
        out_type=jax.ShapeDtypeStruct((BATCH, D_EMB), jnp.float32),
        scratch_types=[
            pltpu.VMEM((ROWS_W,), jnp.int32),
            pltpu.VMEM((ROWS_W, D_EMB), jnp.float32),
            pltpu.SemaphoreType.DMA,
        ],
    )
    def _row_gather(mb_hbm, idx_hbm, out_hbm, idx_v, rows_v, sem):
        wid = lax.axis_index("s") * NC + lax.axis_index("c")
        base = wid * ROWS_W
        pltpu.sync_copy(idx_hbm.at[pl.ds(base, ROWS_W)], idx_v)
        pltpu.async_copy(mb_hbm.at[idx_v], rows_v, sem).wait()
        pltpu.sync_copy(rows_v, out_hbm.at[pl.ds(base, ROWS_W)])

    return _row_gather


def _noise_term(x):
    return LOG_C0 - jnp.log(jnp.exp(x / TAU) / Z + C0E)


@functools.lru_cache(maxsize=None)
def _partial_kernel(size, padded):
    def body(g_ref, out_ref):
        s = jnp.sum(_noise_term(g_ref[...]))
        if padded > size:
            seg = g_ref[pl.ds(0, 128)]
            lane = lax.broadcasted_iota(jnp.int32, (128,), 0)
            v0 = jnp.sum(jnp.where(lane == 0, seg, 0.0))
            s = s - jnp.float32(padded - size) * _noise_term(v0)
        out_ref[...] = jnp.full((1, 128), s, jnp.float32)

    def call(g_out):
        return pl.pallas_call(
            body,
            out_shape=jax.ShapeDtypeStruct((1, 128), jnp.float32),
        )(g_out)

    return call


def _final_body(emb_ref, md_ref, *refs):
    p_refs = refs[:_G]
    entries_ref, sums_ref = refs[_G], refs[_G + 1]
    e = emb_ref[...]
    md = md_ref[...]
    data_ip = jnp.sum(e * md, axis=1)
    dp = jnp.exp(data_ip / TAU) / Z
    sum_ld = jnp.sum(jnp.log(dp) - jnp.log(dp + C0E))
    sum_ln = jnp.float32(0.0)
    lane = lax.broadcasted_iota(jnp.int32, (1, 128), 1)
    for p_ref in p_refs:
        sum_ln = sum_ln + jnp.sum(jnp.where(lane == 0, p_ref[...], 0.0))
    upd = GAMMA * md + (1.0 - GAMMA) * e
    entries_ref[...] = upd / jnp.sqrt(jnp.sum(upd * upd, axis=1,
                                              keepdims=True))
    row = lax.broadcasted_iota(jnp.int32, (8, 128), 0)
    sums_ref[...] = jnp.where(row == 0, sum_ld, sum_ln)


def _final_kernel(emb, mem_data, partials):
    return pl.pallas_call(
        _final_body,
        out_shape=(jax.ShapeDtypeStruct((BATCH, D_EMB), jnp.float32),
                   jax.ShapeDtypeStruct((8, 128), jnp.float32)),
    )(emb, mem_data, *partials)


def kernel(outputs, indices, memory_bank, W, b):
    emb, embh = _emb_kernel(outputs, W, b.reshape(1, D_EMB))
    mem_data = _row_gather_kernel()(memory_bank, indices)
    partials = []
    for g, (loc, size, padded) in enumerate(_GROUPS):
        s_g = _score_kernel(memory_bank, embh, g)
        out_g = _noise_gather_kernel(padded // NW)(s_g, jnp.asarray(loc))
        partials.append(_partial_kernel(size, padded)(out_g))
    entries, sums = _final_kernel(emb, mem_data, partials)
    sum_ld = sums[0, 0]
    sum_ln = sums[1, 0]
    loss = jnp.reshape(-(sum_ld + sum_ln) / BATCH, (1,))
    data_loss = jnp.reshape(-sum_ld / BATCH, (1,))
    noise_loss = jnp.reshape(-sum_ln / BATCH, (1,))
    return (loss, entries, data_loss, noise_loss)


# speedup vs baseline: 1.0954x; 1.0272x over previous
import functools

import numpy as np
import jax
import jax.numpy as jnp
from jax import lax
from jax.experimental import pallas as pl
from jax.experimental.pallas import tpu as pltpu
from jax.experimental.pallas import tpu_sc as plsc

N_TOTAL = 100000
D_MODEL = 2048
D_EMB = 128
BATCH = 1024
M_NOISE = 4096
GAMMA = 0.5
TAU = 0.07
Z = np.float32(2876934.2 / 1281167 * N_TOTAL)
C0E = np.float32(M_NOISE * (1.0 / N_TOTAL) + 1e-07)
LOG_C0 = np.float32(np.log(M_NOISE * (1.0 / N_TOTAL)))

NC, NS = 2, 16
NW = NC * NS
CS = 16384
ROWS_W = BATCH // NW


def _tf2x32(k1, k2, x0, x1):
    k1 = np.uint32(k1)
    k2 = np.uint32(k2)
    ks = (k1, k2, np.uint32(k1 ^ k2 ^ np.uint32(0x1BD11BDA)))
    rot = (np.array([13, 15, 26, 6]), np.array([17, 29, 16, 24]))
    x0 = x0.astype(np.uint32) + ks[0]
    x1 = x1.astype(np.uint32) + ks[1]

    def rl(x, d):
        return (x << np.uint32(d)) | (x >> np.uint32(32 - d))

    for i in range(5):
        for r in rot[i % 2]:
            x0 = x0 + x1
            x1 = rl(x1, r)
            x1 = x0 ^ x1
        x0 = x0 + ks[(i + 1) % 3]
        x1 = x1 + ks[(i + 2) % 3] + np.uint32(i + 1)
    return x0, x1


def _np_randint_fixed_key(shape, n_total, seed=12345):
    size = int(np.prod(shape))
    b1, b2 = _tf2x32(np.uint32(seed >> 32), np.uint32(seed & 0xFFFFFFFF),
                     np.zeros(2, np.uint32), np.arange(2, dtype=np.uint32))
    subkeys = [(b1[i], b2[i]) for i in range(2)]

    def bits(key):
        a, b = _tf2x32(key[0], key[1], np.zeros(size, np.uint32),
                       np.arange(size, dtype=np.uint32))
        return a ^ b

    higher, lower = bits(subkeys[0]), bits(subkeys[1])
    span = np.uint32(n_total)
    with np.errstate(over="ignore"):
        m0 = np.uint32(2 ** 16) % span
        mult = (m0 * m0) % span
        off = ((higher % span) * mult + (lower % span)) % span
    return off.astype(np.int32).reshape(shape)


_RIDX = _np_randint_fixed_key((BATCH, M_NOISE), N_TOTAL)

_G = 4
_KR = N_TOTAL // _G

_k_flat = _RIDX.reshape(-1).astype(np.int64)
_i_flat = np.repeat(np.arange(BATCH, dtype=np.int64), M_NOISE)
_GROUPS = []
for _g in range(_G):
    _sel = (_k_flat >= _g * _KR) & (_k_flat < (_g + 1) * _KR)
    _loc = ((_k_flat[_sel] - _g * _KR) * BATCH
            + _i_flat[_sel]).astype(np.int32)
    _size = int(_loc.size)
    _padded = -(-_size // (NW * 32)) * (NW * 32)
    _loc = np.concatenate(
        [_loc, np.full(_padded - _size, _loc[0], np.int32)])
    _GROUPS.append((_loc, _size, _padded))


def _emb_body(o_ref, w_ref, b_ref, emb_ref, embh_ref):
    x = lax.dot_general(o_ref[...], w_ref[...], (((1,), (1,)), ((), ())),
                        preferred_element_type=jnp.float32,
                        precision=lax.Precision.HIGHEST)
    x = x + b_ref[...]
    e = x / jnp.sqrt(jnp.sum(x * x, axis=1, keepdims=True))
    emb_ref[...] = e
    embh_ref[...] = e.astype(jnp.bfloat16)


def _emb_kernel(outputs, W, b2):
    return pl.pallas_call(
        _emb_body,
        out_shape=(jax.ShapeDtypeStruct((BATCH, D_EMB), jnp.float32),
                   jax.ShapeDtypeStruct((BATCH, D_EMB), jnp.bfloat16)),
    )(outputs, W, b2)


_KB = 1000


def _score_body(mb_ref, embh_ref, s_ref):
    s = lax.dot_general(
        mb_ref[...].astype(jnp.bfloat16), embh_ref[...],
        (((1,), (1,)), ((), ())), preferred_element_type=jnp.float32)
    s_ref[...] = s.reshape(_KB * BATCH)


def _score_kernel(memory_bank, embh, g):
    off = g * (_KR // _KB)
    return pl.pallas_call(
        _score_body,
        grid=(_KR // _KB,),
        in_specs=[pl.BlockSpec((_KB, D_EMB), lambda i: (off + i, 0)),
                  pl.BlockSpec((BATCH, D_EMB), lambda i: (0, 0))],
        out_specs=pl.BlockSpec((_KB * BATCH,), lambda i: (i,)),
        out_shape=jax.ShapeDtypeStruct((_KR * BATCH,), jnp.float32),
    )(memory_bank, embh)


@functools.lru_cache(maxsize=None)
def _noise_gather_kernel(npw):
    fc, tail = divmod(npw, CS)
    mesh = plsc.VectorSubcoreMesh(core_axis_name="c", subcore_axis_name="s")
    scratch = [
        pltpu.VMEM((CS,), jnp.int32),
        pltpu.VMEM((CS,), jnp.float32),
        pltpu.SemaphoreType.DMA,
    ]
    if tail:
        scratch += [pltpu.VMEM((tail,), jnp.int32),
                    pltpu.VMEM((tail,), jnp.float32)]

    @functools.partial(
        pl.kernel,
        mesh=mesh,
        out_type=jax.ShapeDtypeStruct((npw * NW,), jnp.float32),
        scratch_types=scratch,
    )
    def _noise_gather(s_hbm, gidx_hbm, out_hbm, idx_v, val_v, sem,
                      *tail_bufs):
        wid = lax.axis_index("s") * NC + lax.axis_index("c")
        base0 = wid * npw

        def chunk(ci, carry):
            base = base0 + ci * CS
            pltpu.sync_copy(gidx_hbm.at[pl.ds(base, CS)], idx_v)
            pltpu.async_copy(s_hbm.at[idx_v], val_v, sem).wait()
            pltpu.sync_copy(val_v, out_hbm.at[pl.ds(base, CS)])
            return carry

        if fc:
            lax.fori_loop(0, fc, chunk, 0)
        if tail:
            ti, tv = tail_bufs
            base = base0 + fc * CS
            pltpu.sync_copy(gidx_hbm.at[pl.ds(base, tail)], ti)
            pltpu.async_copy(s_hbm.at[ti], tv, sem).wait()
            pltpu.sync_copy(tv, out_hbm.at[pl.ds(base, tail)])

    return _noise_gather


@functools.lru_cache(maxsize=None)
def _row_gather_kernel():
    mesh = plsc.VectorSubcoreMesh(core_axis_name="c", subcore_axis_name="s")

    @functools.partial(
        pl.kernel,
        mesh=mesh,
        out_type=jax.ShapeDtypeStruct((BATCH, D_EMB), jnp.float32),
        scratch_types=[
            pltpu.VMEM((ROWS_W,), jnp.int32),
            pltpu.VMEM((ROWS_W, D_EMB), jnp.float32),
            pltpu.SemaphoreType.DMA,
        ],
    )
    def _row_gather(mb_hbm, idx_hbm, out_hbm, idx_v, rows_v, sem):
        wid = lax.axis_index("s") * NC + lax.axis_index("c")
        base = wid * ROWS_W
        pltpu.sync_copy(idx_hbm.at[pl.ds(base, ROWS_W)], idx_v)
        pltpu.async_copy(mb_hbm.at[idx_v], rows_v, sem).wait()
        pltpu.sync_copy(rows_v, out_hbm.at[pl.ds(base, ROWS_W)])

    return _row_gather


def _noise_term(x):
    return LOG_C0 - jnp.log(jnp.exp(x / TAU) / Z + C0E)


@functools.lru_cache(maxsize=None)
def _partial_kernel(size, padded):
    def body(g_ref, out_ref):
        s = jnp.sum(_noise_term(g_ref[...]))
        if padded > size:
            seg = g_ref[pl.ds(0, 128)]
            lane = lax.broadcasted_iota(jnp.int32, (128,), 0)
            v0 = jnp.sum(jnp.where(lane == 0, seg, 0.0))
            s = s - jnp.float32(padded - size) * _noise_term(v0)
        out_ref[...] = jnp.full((1, 128), s, jnp.float32)

    def call(g_out):
        return pl.pallas_call(
            body,
            out_shape=jax.ShapeDtypeStruct((1, 128), jnp.float32),
        )(g_out)

    return call


def _final_body(emb_ref, md_ref, *refs):
    p_refs = refs[:_G]
    entries_ref, sums_ref = refs[_G], refs[_G + 1]
    e = emb_ref[...]
    md = md_ref[...]
    data_ip = jnp.sum(e * md, axis=1)
    dp = jnp.exp(data_ip / TAU) / Z
    sum_ld = jnp.sum(jnp.log(dp) - jnp.log(dp + C0E))
    sum_ln = jnp.float32(0.0)
    lane = lax.broadcasted_iota(jnp.int32, (1, 128), 1)
    for p_ref in p_refs:
        sum_ln = sum_ln + jnp.sum(jnp.where(lane == 0, p_ref[...], 0.0))
    upd = GAMMA * md + (1.0 - GAMMA) * e
    entries_ref[...] = upd / jnp.sqrt(jnp.sum(upd * upd, axis=1,
                                              keepdims=True))
    row = lax.broadcasted_iota(jnp.int32, (8, 128), 0)
    sums_ref[...] = jnp.where(row == 0, sum_ld, sum_ln)


def _final_kernel(emb, mem_data, partials):
    return pl.pallas_call(
        _final_body,
        out_shape=(jax.ShapeDtypeStruct((BATCH, D_EMB), jnp.float32),
                   jax.ShapeDtypeStruct((8, 128), jnp.float32)),
    )(emb, mem_data, *partials)


def kernel(outputs, indices, memory_bank, W, b):
    emb, embh = _emb_kernel(outputs, W, b.reshape(1, D_EMB))
    mem_data = _row_gather_kernel()(memory_bank, indices)
    partials = []
    for g, (loc, size, padded) in enumerate(_GROUPS):
        s_g = _score_kernel(memory_bank, embh, g)
        out_g = _noise_gather_kernel(padded // NW)(s_g, jnp.asarray(loc))
        partials.append(_partial_kernel(size, padded)(out_g))
    entries, sums = _final_kernel(emb, mem_data, partials)
    sum_ld = sums[0, 0]
    sum_ln = sums[1, 0]
    loss = jnp.reshape(-(sum_ld + sum_ln) / BATCH, (1,))
    data_loss = jnp.reshape(-sum_ld / BATCH, (1,))
    noise_loss = jnp.reshape(-sum_ln / BATCH, (1,))
    return (loss, entries, data_loss, noise_loss)


# speedup vs baseline: 1.1082x; 1.0117x over previous
import functools

import numpy as np
import jax
import jax.numpy as jnp
from jax import lax
from jax.experimental import pallas as pl
from jax.experimental.pallas import tpu as pltpu
from jax.experimental.pallas import tpu_sc as plsc

N_TOTAL = 100000
D_MODEL = 2048
D_EMB = 128
BATCH = 1024
M_NOISE = 4096
GAMMA = 0.5
TAU = 0.07
Z = np.float32(2876934.2 / 1281167 * N_TOTAL)
C0E = np.float32(M_NOISE * (1.0 / N_TOTAL) + 1e-07)
LOG_C0 = np.float32(np.log(M_NOISE * (1.0 / N_TOTAL)))

NC, NS = 2, 16
NW = NC * NS
CS = 32768
ROWS_W = BATCH // NW


def _tf2x32(k1, k2, x0, x1):
    k1 = np.uint32(k1)
    k2 = np.uint32(k2)
    ks = (k1, k2, np.uint32(k1 ^ k2 ^ np.uint32(0x1BD11BDA)))
    rot = (np.array([13, 15, 26, 6]), np.array([17, 29, 16, 24]))
    x0 = x0.astype(np.uint32) + ks[0]
    x1 = x1.astype(np.uint32) + ks[1]

    def rl(x, d):
        return (x << np.uint32(d)) | (x >> np.uint32(32 - d))

    for i in range(5):
        for r in rot[i % 2]:
            x0 = x0 + x1
            x1 = rl(x1, r)
            x1 = x0 ^ x1
        x0 = x0 + ks[(i + 1) % 3]
        x1 = x1 + ks[(i + 2) % 3] + np.uint32(i + 1)
    return x0, x1


def _np_randint_fixed_key(shape, n_total, seed=12345):
    size = int(np.prod(shape))
    b1, b2 = _tf2x32(np.uint32(seed >> 32), np.uint32(seed & 0xFFFFFFFF),
                     np.zeros(2, np.uint32), np.arange(2, dtype=np.uint32))
    subkeys = [(b1[i], b2[i]) for i in range(2)]

    def bits(key):
        a, b = _tf2x32(key[0], key[1], np.zeros(size, np.uint32),
                       np.arange(size, dtype=np.uint32))
        return a ^ b

    higher, lower = bits(subkeys[0]), bits(subkeys[1])
    span = np.uint32(n_total)
    with np.errstate(over="ignore"):
        m0 = np.uint32(2 ** 16) % span
        mult = (m0 * m0) % span
        off = ((higher % span) * mult + (lower % span)) % span
    return off.astype(np.int32).reshape(shape)


_RIDX = _np_randint_fixed_key((BATCH, M_NOISE), N_TOTAL)

_G = 4
_KR = N_TOTAL // _G

_k_flat = _RIDX.reshape(-1).astype(np.int64)
_i_flat = np.repeat(np.arange(BATCH, dtype=np.int64), M_NOISE)
_GROUPS = []
for _g in range(_G):
    _sel = (_k_flat >= _g * _KR) & (_k_flat < (_g + 1) * _KR)
    _loc = ((_k_flat[_sel] - _g * _KR) * BATCH
            + _i_flat[_sel]).astype(np.int32)
    _size = int(_loc.size)
    _padded = -(-_size // (NW * 32)) * (NW * 32)
    _loc = np.concatenate(
        [_loc, np.full(_padded - _size, _loc[0], np.int32)])
    _GROUPS.append((_loc, _size, _padded))


def _emb_body(o_ref, w_ref, b_ref, emb_ref, embh_ref):
    x = lax.dot_general(o_ref[...], w_ref[...], (((1,), (1,)), ((), ())),
                        preferred_element_type=jnp.float32,
                        precision=lax.Precision.HIGHEST)
    x = x + b_ref[...]
    e = x / jnp.sqrt(jnp.sum(x * x, axis=1, keepdims=True))
    emb_ref[...] = e
    embh_ref[...] = e.astype(jnp.bfloat16)


def _emb_kernel(outputs, W, b2):
    return pl.pallas_call(
        _emb_body,
        out_shape=(jax.ShapeDtypeStruct((BATCH, D_EMB), jnp.float32),
                   jax.ShapeDtypeStruct((BATCH, D_EMB), jnp.bfloat16)),
    )(outputs, W, b2)


_KB = 1000


def _score_body(mb_ref, embh_ref, s_ref):
    s = lax.dot_general(
        mb_ref[...].astype(jnp.bfloat16), embh_ref[...],
        (((1,), (1,)), ((), ())), preferred_element_type=jnp.float32)
    s_ref[...] = s.reshape(_KB * BATCH)


def _score_kernel(memory_bank, embh, g):
    off = g * (_KR // _KB)
    return pl.pallas_call(
        _score_body,
        grid=(_KR // _KB,),
        in_specs=[pl.BlockSpec((_KB, D_EMB), lambda i: (off + i, 0)),
                  pl.BlockSpec((BATCH, D_EMB), lambda i: (0, 0))],
        out_specs=pl.BlockSpec((_KB * BATCH,), lambda i: (i,)),
        out_shape=jax.ShapeDtypeStruct((_KR * BATCH,), jnp.float32),
    )(memory_bank, embh)


@functools.lru_cache(maxsize=None)
def _noise_gather_kernel(npw):
    fc, tail = divmod(npw, CS)
    mesh = plsc.VectorSubcoreMesh(core_axis_name="c", subcore_axis_name="s")
    scratch = [
        pltpu.VMEM((CS,), jnp.int32),
        pltpu.VMEM((CS,), jnp.float32),
        pltpu.SemaphoreType.DMA,
    ]
    if tail:
        scratch += [pltpu.VMEM((tail,), jnp.int32),
                    pltpu.VMEM((tail,), jnp.float32)]

    @functools.partial(
        pl.kernel,
        mesh=mesh,
        out_type=jax.ShapeDtypeStruct((npw * NW,), jnp.float32),
        scratch_types=scratch,
    )
    def _noise_gather(s_hbm, gidx_hbm, out_hbm, idx_v, val_v, sem,
                      *tail_bufs):
        wid = lax.axis_index("s") * NC + lax.axis_index("c")
        base0 = wid * npw

        def chunk(ci, carry):
            base = base0 + ci * CS
            pltpu.sync_copy(gidx_hbm.at[pl.ds(base, CS)], idx_v)
            pltpu.async_copy(s_hbm.at[idx_v], val_v, sem).wait()
            pltpu.sync_copy(val_v, out_hbm.at[pl.ds(base, CS)])
            return carry

        if fc:
            lax.fori_loop(0, fc, chunk, 0)
        if tail:
            ti, tv = tail_bufs
            base = base0 + fc * CS
            pltpu.sync_copy(gidx_hbm.at[pl.ds(base, tail)], ti)
            pltpu.async_copy(s_hbm.at[ti], tv, sem).wait()
            pltpu.sync_copy(tv, out_hbm.at[pl.ds(base, tail)])

    return _noise_gather


@functools.lru_cache(maxsize=None)
def _row_gather_kernel():
    mesh = plsc.VectorSubcoreMesh(core_axis_name="c", subcore_axis_name="s")

    @functools.partial(
        pl.kernel,
        mesh=mesh,
        out_type=jax.ShapeDtypeStruct((BATCH, D_EMB), jnp.float32),
        scratch_types=[
            pltpu.VMEM((ROWS_W,), jnp.int32),
            pltpu.VMEM((ROWS_W, D_EMB), jnp.float32),
            pltpu.SemaphoreType.DMA,
        ],
    )
    def _row_gather(mb_hbm, idx_hbm, out_hbm, idx_v, rows_v, sem):
        wid = lax.axis_index("s") * NC + lax.axis_index("c")
        base = wid * ROWS_W
        pltpu.sync_copy(idx_hbm.at[pl.ds(base, ROWS_W)], idx_v)
        pltpu.async_copy(mb_hbm.at[idx_v], rows_v, sem).wait()
        pltpu.sync_copy(rows_v, out_hbm.at[pl.ds(base, ROWS_W)])

    return _row_gather


def _noise_term(x):
    return LOG_C0 - jnp.log(jnp.exp(x / TAU) / Z + C0E)


@functools.lru_cache(maxsize=None)
def _partial_kernel(size, padded):
    def body(g_ref, out_ref):
        s = jnp.sum(_noise_term(g_ref[...]))
        if padded > size:
            seg = g_ref[pl.ds(0, 128)]
            lane = lax.broadcasted_iota(jnp.int32, (128,), 0)
            v0 = jnp.sum(jnp.where(lane == 0, seg, 0.0))
            s = s - jnp.float32(padded - size) * _noise_term(v0)
        out_ref[...] = jnp.full((1, 128), s, jnp.float32)

    def call(g_out):
        return pl.pallas_call(
            body,
            out_shape=jax.ShapeDtypeStruct((1, 128), jnp.float32),
        )(g_out)

    return call


def _final_body(emb_ref, md_ref, *refs):
    p_refs = refs[:_G]
    entries_ref, sums_ref = refs[_G], refs[_G + 1]
    e = emb_ref[...]
    md = md_ref[...]
    data_ip = jnp.sum(e * md, axis=1)
    dp = jnp.exp(data_ip / TAU) / Z
    sum_ld = jnp.sum(jnp.log(dp) - jnp.log(dp + C0E))
    sum_ln = jnp.float32(0.0)
    lane = lax.broadcasted_iota(jnp.int32, (1, 128), 1)
    for p_ref in p_refs:
        sum_ln = sum_ln + jnp.sum(jnp.where(lane == 0, p_ref[...], 0.0))
    upd = GAMMA * md + (1.0 - GAMMA) * e
    entries_ref[...] = upd / jnp.sqrt(jnp.sum(upd * upd, axis=1,
                                              keepdims=True))
    row = lax.broadcasted_iota(jnp.int32, (8, 128), 0)
    sums_ref[...] = jnp.where(row == 0, sum_ld, sum_ln)


def _final_kernel(emb, mem_data, partials):
    return pl.pallas_call(
        _final_body,
        out_shape=(jax.ShapeDtypeStruct((BATCH, D_EMB), jnp.float32),
                   jax.ShapeDtypeStruct((8, 128), jnp.float32)),
    )(emb, mem_data, *partials)


def kernel(outputs, indices, memory_bank, W, b):
    emb, embh = _emb_kernel(outputs, W, b.reshape(1, D_EMB))
    mem_data = _row_gather_kernel()(memory_bank, indices)
    partials = []
    for g, (loc, size, padded) in enumerate(_GROUPS):
        s_g = _score_kernel(memory_bank, embh, g)
        out_g = _noise_gather_kernel(padded // NW)(s_g, jnp.asarray(loc))
        partials.append(_partial_kernel(size, padded)(out_g))
    entries, sums = _final_kernel(emb, mem_data, partials)
    sum_ld = sums[0, 0]
    sum_ln = sums[1, 0]
    loss = jnp.reshape(-(sum_ld + sum_ln) / BATCH, (1,))
    data_loss = jnp.reshape(-sum_ld / BATCH, (1,))
    noise_loss = jnp.reshape(-sum_ln / BATCH, (1,))
    return (loss, entries, data_loss, noise_loss)
